# baseline (speedup 1.0000x reference)
import functools

import jax
import jax.numpy as jnp
from jax import lax
from jax.experimental import pallas as pl
from jax.experimental.pallas import tpu as pltpu
from jax.experimental.pallas import tpu_sc as plsc

F32 = jnp.float32
CITY = 25000
GROUP = 16
NB = 2
N = 2 * CITY
H = 64
XE = 32
EH = 16
NGE = 512
E = 800000

EC = 128
NSUB = 16
NBUF = 3
NCHUNK = 393
EPAD = NSUB * NCHUNK * EC
NCHT = NSUB * NCHUNK
NPAD = 50048
RB = NPAD // NSUB
NHALF = 25024
NPH = 25056
RBH = NPH // NSUB

RBLK = 5000
NBLK = N // RBLK

_PREC = lax.Precision.HIGHEST


def _mm(a, b):
    return jnp.dot(a, b, precision=_PREC)


def _mm_t(a, b):
    return lax.dot_general(a, b, (((0,), (0,)), ((), ())), precision=_PREC)


def _k1a_body(x_ref, we_ref, be_ref, tw_ref, h_ref, g_ref):
    i = pl.program_id(0)
    h = _mm(x_ref[...], we_ref[...]) + be_ref[...]
    h_ref[...] = h
    gp = _mm_t(tw_ref[...], h)[None]

    @pl.when(i % (NBLK // NB) == 0)
    def _():
        g_ref[...] = jnp.zeros_like(g_ref)

    g_ref[...] += gp


def _k1a(x, we, be2, trans_w):
    return pl.pallas_call(
        _k1a_body,
        grid=(NBLK,),
        in_specs=[
            pl.BlockSpec((RBLK, H), lambda i: (i, 0)),
            pl.BlockSpec((H, XE), lambda i: (0, 0)),
            pl.BlockSpec((1, XE), lambda i: (0, 0)),
            pl.BlockSpec((RBLK, GROUP), lambda i: (i % (NBLK // NB), 0)),
        ],
        out_specs=[
            pl.BlockSpec((RBLK, XE), lambda i: (i, 0)),
            pl.BlockSpec((1, GROUP, XE), lambda i: (i // (NBLK // NB), 0, 0)),
        ],
        out_shape=[
            jax.ShapeDtypeStruct((N, XE), F32),
            jax.ShapeDtypeStruct((NB, GROUP, XE), F32),
        ],
    )(x, we, be2, trans_w)


def _k1b_body(g_ref, gei_ref, gew_ref,
              w1a0_ref, w1e0_ref, b10_ref, w2a0_ref, w2b0_ref, b20_ref,
              w1a1_ref, w1e1_ref, b11_ref, w2a1_ref, w2b1_ref, b21_ref,
              out_ref):
    ng = NB * GROUP
    row = gei_ref[0:1, :]
    col = gei_ref[1:2, :]
    iot = lax.broadcasted_iota(jnp.int32, (ng, NGE), 0)
    rowoh = (iot == jnp.broadcast_to(row, (ng, NGE))).astype(F32)
    coloh = (iot == jnp.broadcast_to(col, (ng, NGE))).astype(F32)
    cnt = jnp.sum(coloh, axis=1, keepdims=True)
    inv = 1.0 / jnp.maximum(cnt, 1.0)
    gew = gew_ref[...]

    def layer(xg, w1a, w1e, b1, w2a, w2b, b2):
        xr = _mm_t(rowoh, xg)
        m = jnp.maximum(_mm(xr, w1a) + _mm(gew, w1e) + b1, 0.0)
        mean = _mm(coloh, m) * inv
        return jnp.maximum(_mm(xg, w2a) + _mm(mean, w2b) + b2, 0.0)

    xg = layer(g_ref[...], w1a0_ref[...], w1e0_ref[...], b10_ref[...],
               w2a0_ref[...], w2b0_ref[...], b20_ref[...])
    out_ref[...] = layer(xg, w1a1_ref[...], w1e1_ref[...], b11_ref[...],
                         w2a1_ref[...], w2b1_ref[...], b21_ref[...])


def _k1b(g32, gei, gew, *weights):
    return pl.pallas_call(
        _k1b_body,
        out_shape=jax.ShapeDtypeStruct((NB * GROUP, H), F32),
    )(g32, gei, gew, *weights)


def _k1c_body(h_ref, tw_ref, g_ref, w1ha_ref, w1hb_ref, w1na_ref, w1nb_ref,
              b1a_ref, b1b_ref, nb_ref, ya_ref, yb_ref):
    hb = h_ref[...]
    nb = _mm(tw_ref[...], g_ref[0])
    nb_ref[...] = nb
    ya_ref[...] = _mm(hb, w1ha_ref[...]) + _mm(nb, w1na_ref[...]) + b1a_ref[...]
    yb_ref[...] = _mm(hb, w1hb_ref[...]) + _mm(nb, w1nb_ref[...]) + b1b_ref[...]


def _k1c(h, trans_w, g_out, w1h, w1n, b12):
    halves = (w1h[:, :32], w1h[:, 32:], w1n[:, :32], w1n[:, 32:],
              b12[:, :32], b12[:, 32:])
    return pl.pallas_call(
        _k1c_body,
        grid=(NBLK,),
        in_specs=[
            pl.BlockSpec((RBLK, XE), lambda i: (i, 0)),
            pl.BlockSpec((RBLK, GROUP), lambda i: (i % (NBLK // NB), 0)),
            pl.BlockSpec((1, GROUP, H), lambda i: (i // (NBLK // NB), 0, 0)),
            pl.BlockSpec((XE, 32), lambda i: (0, 0)),
            pl.BlockSpec((XE, 32), lambda i: (0, 0)),
            pl.BlockSpec((H, 32), lambda i: (0, 0)),
            pl.BlockSpec((H, 32), lambda i: (0, 0)),
            pl.BlockSpec((1, 32), lambda i: (0, 0)),
            pl.BlockSpec((1, 32), lambda i: (0, 0)),
        ],
        out_specs=[
            pl.BlockSpec((RBLK, H), lambda i: (i, 0)),
            pl.BlockSpec((RBLK, 32), lambda i: (i, 0)),
            pl.BlockSpec((RBLK, 32), lambda i: (i, 0)),
        ],
        out_shape=[
            jax.ShapeDtypeStruct((N, H), F32),
            jax.ShapeDtypeStruct((N, 32), F32),
            jax.ShapeDtypeStruct((N, 32), F32),
        ],
    )(h, trans_w, g_out, *halves)


def _sc_edge_body(ya, yb, ei3, vvec, zrows, zcnt, ones_h,
                  outa, outb, cnta_out, cntb_out,
                  acc_sh, cnt_sh, vseg_v, ones_v,
                  idx0, idx1, idx2,
                  cidx0, cidx1, cidx2,
                  rows0, rows1, rows2,
                  sg0, sg1, sg2, ss0, ss1, ss2,
                  sc0, sc1, sc2):
    c = lax.axis_index("c")
    s = lax.axis_index("s")
    r0 = s * RB
    idx = (idx0, idx1, idx2)
    cidx = (cidx0, cidx1, cidx2)
    rows = (rows0, rows1, rows2)
    sg = (sg0, sg1, sg2)
    ss = (ss0, ss1, ss2)
    sc = (sc0, sc1, sc2)
    cbase = c * NHALF

    pltpu.sync_copy(zrows.at[pl.ds(r0, RB)], acc_sh.at[pl.ds(r0, RB)])
    pltpu.sync_copy(zcnt.at[pl.ds(s * RBH, RBH)],
                    cnt_sh.at[pl.ds(s * RBH, RBH)])

    pltpu.sync_copy(vvec.at[pl.ds(c * 32, 32)], vseg_v)
    pltpu.sync_copy(ones_h, ones_v)

    plsc.subcore_barrier()

    vs0 = vseg_v[pl.ds(0, 16)]
    vs1 = vseg_v[pl.ds(16, 16)]

    def idx_load(b, j):
        pltpu.sync_copy(ei3.at[s * NCHUNK + j], idx[b])

    def gather_start(b):
        @pl.when(c == 0)
        def _():
            pltpu.async_copy(ya.at[idx[b].at[0]], rows[b], sg[b])

        @pl.when(c == 1)
        def _():
            pltpu.async_copy(yb.at[idx[b].at[0]], rows[b], sg[b])

    def gather_wait(b):
        @pl.when(c == 0)
        def _():
            pltpu.make_async_copy(ya.at[idx[b].at[0]], rows[b], sg[b]).wait()

        @pl.when(c == 1)
        def _():
            pltpu.make_async_copy(yb.at[idx[b].at[0]], rows[b], sg[b]).wait()

    def scatter_start(b):
        pltpu.async_copy(rows[b], acc_sh.at[idx[b].at[1]], ss[b], add=True)
        pltpu.async_copy(ones_v, cnt_sh.at[cidx[b]], sc[b], add=True)

    def scatter_wait(b):
        pltpu.make_async_copy(rows[b], acc_sh.at[idx[b].at[1]], ss[b]).wait()
        pltpu.make_async_copy(ones_v, cnt_sh.at[cidx[b]], sc[b]).wait()

    def compute(b):
        rb_ = rows[b]
        ib_ = idx[b]
        cb_ = cidx[b]

        def edge_group(g, carry):
            sl16 = pl.ds(g * 16, 16)
            ci = ib_[1, sl16] - cbase
            cb_[sl16] = jnp.where(
                jnp.logical_and(ci >= 0, ci < NHALF), ci, NHALF)
            ew16 = plsc.bitcast(ib_[2, sl16], F32)
            for k in range(16):
                e = g * 16 + k
                w = ew16[k]
                rb_[e, pl.ds(0, 16)] = jnp.maximum(
                    rb_[e, pl.ds(0, 16)] + w * vs0, 0.0)
                rb_[e, pl.ds(16, 16)] = jnp.maximum(
                    rb_[e, pl.ds(16, 16)] + w * vs1, 0.0)
            return carry

        lax.fori_loop(0, EC // 16, edge_group, 0)

    for b in range(NBUF - 1):
        idx_load(b, b)
        gather_start(b)

    def outer(g, carry):
        for b in range(NBUF):
            j = g * NBUF + b
            jn = j + (NBUF - 1)
            bn = (b + NBUF - 1) % NBUF
            gather_wait(b)
            compute(b)
            scatter_start(b)

            @pl.when(jn < NCHUNK)
            def _():
                @pl.when(j >= 1)
                def _():
                    scatter_wait(bn)

                idx_load(bn, jn)
                gather_start(bn)

        return carry

    lax.fori_loop(0, NCHUNK // NBUF, outer, 0)

    for b in range(NBUF):
        scatter_wait(b)

    plsc.subcore_barrier()

    @pl.when(c == 0)
    def _():
        pltpu.sync_copy(acc_sh.at[pl.ds(r0, RB)], outa.at[pl.ds(r0, RB)])
        pltpu.sync_copy(cnt_sh.at[pl.ds(s * RBH, RBH)],
                        cnta_out.at[pl.ds(s * RBH, RBH)])

    @pl.when(c == 1)
    def _():
        pltpu.sync_copy(acc_sh.at[pl.ds(r0, RB)], outb.at[pl.ds(r0, RB)])
        pltpu.sync_copy(cnt_sh.at[pl.ds(s * RBH, RBH)],
                        cntb_out.at[pl.ds(s * RBH, RBH)])


_sc_edge_pass = functools.partial(
    pl.kernel,
    out_type=(
        jax.ShapeDtypeStruct((NPAD, 32), F32),
        jax.ShapeDtypeStruct((NPAD, 32), F32),
        jax.ShapeDtypeStruct((NPH, 8), F32),
        jax.ShapeDtypeStruct((NPH, 8), F32),
    ),
    mesh=plsc.VectorSubcoreMesh(core_axis_name="c", subcore_axis_name="s",
                                num_cores=2, num_subcores=NSUB),
    compiler_params=pltpu.CompilerParams(use_tc_tiling_on_sc=False,
                                         needs_layout_passes=False),
    scratch_types=(
        [pltpu.VMEM_SHARED((NPAD, 32), F32),
         pltpu.VMEM_SHARED((NPH, 8), F32),
         pltpu.VMEM((32,), F32),
         pltpu.VMEM((EC, 8), F32)]
        + [pltpu.VMEM((3, EC), jnp.int32) for _ in range(NBUF)]
        + [pltpu.VMEM((EC,), jnp.int32) for _ in range(NBUF)]
        + [pltpu.VMEM((EC, 32), F32) for _ in range(NBUF)]
        + [pltpu.SemaphoreType.DMA for _ in range(3 * NBUF)]
    ),
)(_sc_edge_body)


def _post0_body(h_ref, nb_ref, aa_ref, ab_ref, cnt_ref,
                w2h_ref, w2n_ref, w2ba_ref, w2bb_ref, b2_ref,
                w1a_ref, w1b_ref, b1a_ref, b1b_ref,
                out_ref, ya_ref, yb_ref):
    inv = 1.0 / jnp.maximum(cnt_ref[...], 1.0)
    z = _mm(aa_ref[...], w2ba_ref[...]) + _mm(ab_ref[...], w2bb_ref[...])
    out = jnp.maximum(
        _mm(h_ref[...], w2h_ref[...]) + _mm(nb_ref[...], w2n_ref[...])
        + z * inv + b2_ref[...], 0.0)
    out_ref[...] = out
    ya_ref[...] = _mm(out, w1a_ref[...]) + b1a_ref[...]
    yb_ref[...] = _mm(out, w1b_ref[...]) + b1b_ref[...]


def _post0(h, nb, acc_a, acc_b, cnt2, w2a, w2b, b22, w1, b12):
    args = (h, nb, acc_a, acc_b, cnt2,
            w2a[:XE], w2a[XE:], w2b[:32], w2b[32:], b22,
            w1[:, :32], w1[:, 32:], b12[:, :32], b12[:, 32:])
    in_specs = [
        pl.BlockSpec((RBLK, XE), lambda i: (i, 0)),
        pl.BlockSpec((RBLK, H), lambda i: (i, 0)),
        pl.BlockSpec((RBLK, 32), lambda i: (i, 0)),
        pl.BlockSpec((RBLK, 32), lambda i: (i, 0)),
        pl.BlockSpec((RBLK, 1), lambda i: (i, 0)),
        pl.BlockSpec((XE, H), lambda i: (0, 0)),
        pl.BlockSpec((H, H), lambda i: (0, 0)),
        pl.BlockSpec((32, H), lambda i: (0, 0)),
        pl.BlockSpec((32, H), lambda i: (0, 0)),
        pl.BlockSpec((1, H), lambda i: (0, 0)),
        pl.BlockSpec((H, 32), lambda i: (0, 0)),
        pl.BlockSpec((H, 32), lambda i: (0, 0)),
        pl.BlockSpec((1, 32), lambda i: (0, 0)),
        pl.BlockSpec((1, 32), lambda i: (0, 0)),
    ]
    out_specs = [
        pl.BlockSpec((RBLK, H), lambda i: (i, 0)),
        pl.BlockSpec((RBLK, 32), lambda i: (i, 0)),
        pl.BlockSpec((RBLK, 32), lambda i: (i, 0)),
    ]
    out_shape = [
        jax.ShapeDtypeStruct((N, H), F32),
        jax.ShapeDtypeStruct((N, 32), F32),
        jax.ShapeDtypeStruct((N, 32), F32),
    ]
    return pl.pallas_call(
        _post0_body, grid=(NBLK,), in_specs=in_specs, out_specs=out_specs,
        out_shape=out_shape,
    )(*args)


def _post1_body(x_ref, aa_ref, ab_ref, cnt_ref,
                w2a_ref, w2ba_ref, w2bb_ref, b2_ref, out_ref):
    inv = 1.0 / jnp.maximum(cnt_ref[...], 1.0)
    z = _mm(aa_ref[...], w2ba_ref[...]) + _mm(ab_ref[...], w2bb_ref[...])
    out_ref[...] = jnp.maximum(
        _mm(x_ref[...], w2a_ref[...]) + z * inv + b2_ref[...], 0.0)


def _post1(x, acc_a, acc_b, cnt2, w2a, w2b, b22):
    args = (x, acc_a, acc_b, cnt2, w2a, w2b[:32], w2b[32:], b22)
    in_specs = [
        pl.BlockSpec((RBLK, H), lambda i: (i, 0)),
        pl.BlockSpec((RBLK, 32), lambda i: (i, 0)),
        pl.BlockSpec((RBLK, 32), lambda i: (i, 0)),
        pl.BlockSpec((RBLK, 1), lambda i: (i, 0)),
        pl.BlockSpec((H, H), lambda i: (0, 0)),
        pl.BlockSpec((32, H), lambda i: (0, 0)),
        pl.BlockSpec((32, H), lambda i: (0, 0)),
        pl.BlockSpec((1, H), lambda i: (0, 0)),
    ]
    return pl.pallas_call(
        _post1_body, grid=(NBLK,), in_specs=in_specs,
        out_specs=pl.BlockSpec((RBLK, H), lambda i: (i, 0)),
        out_shape=jax.ShapeDtypeStruct((N, H), F32),
    )(*args)


def kernel(x, trans_w, g_edge_index, g_edge_w, edge_index, edge_w, params):
    p = params
    we = p['we']
    be2 = p['be'].reshape(1, XE)
    g0, g1 = p['group']
    gl0, gl1 = p['global']

    gw = (
        g0['mlp1']['w'][:XE], g0['mlp1']['w'][XE:], g0['mlp1']['b'].reshape(1, H),
        g0['mlp2']['w'][:XE], g0['mlp2']['w'][XE:], g0['mlp2']['b'].reshape(1, H),
        g1['mlp1']['w'][:H], g1['mlp1']['w'][H:], g1['mlp1']['b'].reshape(1, H),
        g1['mlp2']['w'][:H], g1['mlp2']['w'][H:], g1['mlp2']['b'].reshape(1, H),
    )
    w1g0 = gl0['mlp1']['w']
    w1h0, w1n0 = w1g0[:XE], w1g0[XE:XE + H]
    v0 = w1g0[XE + H]
    b1g0 = gl0['mlp1']['b'].reshape(1, H)
    w2a0, w2b0 = gl0['mlp2']['w'][:XE + H], gl0['mlp2']['w'][XE + H:]
    b2g0 = gl0['mlp2']['b'].reshape(1, H)
    w1g1 = gl1['mlp1']['w']
    w1a1, v1 = w1g1[:H], w1g1[H]
    b1g1 = gl1['mlp1']['b'].reshape(1, H)
    w2a1, w2b1 = gl1['mlp2']['w'][:H], gl1['mlp2']['w'][H:]
    b2g1 = gl1['mlp2']['b'].reshape(1, H)

    npd = EPAD - E
    erow = jnp.concatenate([edge_index[0], jnp.zeros((npd,), jnp.int32)])
    ecol = jnp.concatenate([edge_index[1], jnp.full((npd,), N, jnp.int32)])
    ewf = jnp.concatenate([edge_w[:, 0], jnp.zeros((npd,), F32)])
    ewbits = lax.bitcast_convert_type(ewf, jnp.int32)
    ei3 = jnp.stack([erow.reshape(NCHT, EC), ecol.reshape(NCHT, EC),
                     ewbits.reshape(NCHT, EC)], axis=1)
    zrows = jnp.zeros((NPAD, 32), F32)
    zcnt = jnp.zeros((NPH, 8), F32)
    ones8 = jnp.ones((EC, 8), F32)

    h, g = _k1a(x, we, be2, trans_w)
    g_out = _k1b(g.reshape(NB * GROUP, XE), g_edge_index, g_edge_w, *gw)
    nb, y0a, y0b = _k1c(h, trans_w, g_out.reshape(NB, GROUP, H),
                        w1h0, w1n0, b1g0)

    acc_a, acc_b, cnta, cntb = _sc_edge_pass(y0a, y0b, ei3, v0,
                                             zrows, zcnt, ones8)
    cnt2 = jnp.concatenate([cnta[:NHALF, 0],
                            cntb[:N - NHALF, 0]]).reshape(N, 1)
    out0, y1a, y1b = _post0(h, nb, acc_a[:N], acc_b[:N], cnt2,
                            w2a0, w2b0, b2g0, w1a1, b1g1)

    acc_a1, acc_b1, _, _ = _sc_edge_pass(y1a, y1b, ei3, v1,
                                         zrows, zcnt, ones8)
    return _post1(out0, acc_a1[:N], acc_b1[:N], cnt2, w2a1, w2b1, b2g1)


# speedup vs baseline: 1.2800x; 1.2800x over previous
import functools

import jax
import jax.numpy as jnp
from jax import lax
from jax.experimental import pallas as pl
from jax.experimental.pallas import tpu as pltpu
from jax.experimental.pallas import tpu_sc as plsc

F32 = jnp.float32
CITY = 25000
GROUP = 16
NB = 2
N = 2 * CITY
H = 64
XE = 32
EH = 16
NGE = 512
E = 800000

EC = 128
NSUB = 16
NBUF = 3
NCHUNK = 393
EPAD = NSUB * NCHUNK * EC
NCHT = NSUB * NCHUNK
NPAD = 50048
RB = NPAD // NSUB
NHALF = 25024
NPH = 25056
RBH = NPH // NSUB

RBLK = 5000
NBLK = N // RBLK

_PREC = lax.Precision.DEFAULT


def _mm(a, b):
    return jnp.dot(a, b, precision=_PREC)


def _mm_t(a, b):
    return lax.dot_general(a, b, (((0,), (0,)), ((), ())), precision=_PREC)


def _k1a_body(x_ref, we_ref, be_ref, tw_ref, h_ref, g_ref):
    i = pl.program_id(0)
    h = _mm(x_ref[...], we_ref[...]) + be_ref[...]
    h_ref[...] = h
    gp = _mm_t(tw_ref[...], h)[None]

    @pl.when(i % (NBLK // NB) == 0)
    def _():
        g_ref[...] = jnp.zeros_like(g_ref)

    g_ref[...] += gp


def _k1a(x, we, be2, trans_w):
    return pl.pallas_call(
        _k1a_body,
        grid=(NBLK,),
        in_specs=[
            pl.BlockSpec((RBLK, H), lambda i: (i, 0)),
            pl.BlockSpec((H, XE), lambda i: (0, 0)),
            pl.BlockSpec((1, XE), lambda i: (0, 0)),
            pl.BlockSpec((RBLK, GROUP), lambda i: (i % (NBLK // NB), 0)),
        ],
        out_specs=[
            pl.BlockSpec((RBLK, XE), lambda i: (i, 0)),
            pl.BlockSpec((1, GROUP, XE), lambda i: (i // (NBLK // NB), 0, 0)),
        ],
        out_shape=[
            jax.ShapeDtypeStruct((N, XE), F32),
            jax.ShapeDtypeStruct((NB, GROUP, XE), F32),
        ],
    )(x, we, be2, trans_w)


def _k1b_body(g_ref, gei_ref, gew_ref,
              w1a0_ref, w1e0_ref, b10_ref, w2a0_ref, w2b0_ref, b20_ref,
              w1a1_ref, w1e1_ref, b11_ref, w2a1_ref, w2b1_ref, b21_ref,
              out_ref):
    ng = NB * GROUP
    row = gei_ref[0:1, :]
    col = gei_ref[1:2, :]
    iot = lax.broadcasted_iota(jnp.int32, (ng, NGE), 0)
    rowoh = (iot == jnp.broadcast_to(row, (ng, NGE))).astype(F32)
    coloh = (iot == jnp.broadcast_to(col, (ng, NGE))).astype(F32)
    cnt = jnp.sum(coloh, axis=1, keepdims=True)
    inv = 1.0 / jnp.maximum(cnt, 1.0)
    gew = gew_ref[...]

    def layer(xg, w1a, w1e, b1, w2a, w2b, b2):
        xr = _mm_t(rowoh, xg)
        m = jnp.maximum(_mm(xr, w1a) + _mm(gew, w1e) + b1, 0.0)
        mean = _mm(coloh, m) * inv
        return jnp.maximum(_mm(xg, w2a) + _mm(mean, w2b) + b2, 0.0)

    xg = layer(g_ref[...], w1a0_ref[...], w1e0_ref[...], b10_ref[...],
               w2a0_ref[...], w2b0_ref[...], b20_ref[...])
    out_ref[...] = layer(xg, w1a1_ref[...], w1e1_ref[...], b11_ref[...],
                         w2a1_ref[...], w2b1_ref[...], b21_ref[...])


def _k1b(g32, gei, gew, *weights):
    return pl.pallas_call(
        _k1b_body,
        out_shape=jax.ShapeDtypeStruct((NB * GROUP, H), F32),
    )(g32, gei, gew, *weights)


def _k1c_body(h_ref, tw_ref, g_ref, w1ha_ref, w1hb_ref, w1na_ref, w1nb_ref,
              b1a_ref, b1b_ref, nb_ref, ya_ref, yb_ref):
    hb = h_ref[...]
    nb = _mm(tw_ref[...], g_ref[0])
    nb_ref[...] = nb
    ya_ref[...] = _mm(hb, w1ha_ref[...]) + _mm(nb, w1na_ref[...]) + b1a_ref[...]
    yb_ref[...] = _mm(hb, w1hb_ref[...]) + _mm(nb, w1nb_ref[...]) + b1b_ref[...]


def _k1c(h, trans_w, g_out, w1h, w1n, b12):
    halves = (w1h[:, :32], w1h[:, 32:], w1n[:, :32], w1n[:, 32:],
              b12[:, :32], b12[:, 32:])
    return pl.pallas_call(
        _k1c_body,
        grid=(NBLK,),
        in_specs=[
            pl.BlockSpec((RBLK, XE), lambda i: (i, 0)),
            pl.BlockSpec((RBLK, GROUP), lambda i: (i % (NBLK // NB), 0)),
            pl.BlockSpec((1, GROUP, H), lambda i: (i // (NBLK // NB), 0, 0)),
            pl.BlockSpec((XE, 32), lambda i: (0, 0)),
            pl.BlockSpec((XE, 32), lambda i: (0, 0)),
            pl.BlockSpec((H, 32), lambda i: (0, 0)),
            pl.BlockSpec((H, 32), lambda i: (0, 0)),
            pl.BlockSpec((1, 32), lambda i: (0, 0)),
            pl.BlockSpec((1, 32), lambda i: (0, 0)),
        ],
        out_specs=[
            pl.BlockSpec((RBLK, H), lambda i: (i, 0)),
            pl.BlockSpec((RBLK, 32), lambda i: (i, 0)),
            pl.BlockSpec((RBLK, 32), lambda i: (i, 0)),
        ],
        out_shape=[
            jax.ShapeDtypeStruct((N, H), F32),
            jax.ShapeDtypeStruct((N, 32), F32),
            jax.ShapeDtypeStruct((N, 32), F32),
        ],
    )(h, trans_w, g_out, *halves)


def _sc_edge_body(ya, yb, ei3, vvec, zrows, zcnt, ones_h,
                  outa, outb, cnta_out, cntb_out,
                  acc_sh, cnt_sh, vseg_v, ones_v,
                  idx0, idx1, idx2,
                  cidx0, cidx1, cidx2,
                  rows0, rows1, rows2,
                  sg0, sg1, sg2, ss0, ss1, ss2,
                  sc0, sc1, sc2):
    c = lax.axis_index("c")
    s = lax.axis_index("s")
    r0 = s * RB
    idx = (idx0, idx1, idx2)
    cidx = (cidx0, cidx1, cidx2)
    rows = (rows0, rows1, rows2)
    sg = (sg0, sg1, sg2)
    ss = (ss0, ss1, ss2)
    sc = (sc0, sc1, sc2)
    cbase = c * NHALF

    pltpu.sync_copy(zrows.at[pl.ds(r0, RB)], acc_sh.at[pl.ds(r0, RB)])
    pltpu.sync_copy(zcnt.at[pl.ds(s * RBH, RBH)],
                    cnt_sh.at[pl.ds(s * RBH, RBH)])

    pltpu.sync_copy(vvec.at[pl.ds(c * 32, 32)], vseg_v)
    pltpu.sync_copy(ones_h, ones_v)

    plsc.subcore_barrier()

    vs0 = vseg_v[pl.ds(0, 16)]
    vs1 = vseg_v[pl.ds(16, 16)]

    def idx_load(b, j):
        pltpu.sync_copy(ei3.at[s * NCHUNK + j], idx[b])

    def gather_start(b):
        @pl.when(c == 0)
        def _():
            pltpu.async_copy(ya.at[idx[b].at[0]], rows[b], sg[b])

        @pl.when(c == 1)
        def _():
            pltpu.async_copy(yb.at[idx[b].at[0]], rows[b], sg[b])

    def gather_wait(b):
        @pl.when(c == 0)
        def _():
            pltpu.make_async_copy(ya.at[idx[b].at[0]], rows[b], sg[b]).wait()

        @pl.when(c == 1)
        def _():
            pltpu.make_async_copy(yb.at[idx[b].at[0]], rows[b], sg[b]).wait()

    def scatter_start(b):
        pltpu.async_copy(rows[b], acc_sh.at[idx[b].at[1]], ss[b], add=True)
        pltpu.async_copy(ones_v, cnt_sh.at[cidx[b]], sc[b], add=True)

    def scatter_wait(b):
        pltpu.make_async_copy(rows[b], acc_sh.at[idx[b].at[1]], ss[b]).wait()
        pltpu.make_async_copy(ones_v, cnt_sh.at[cidx[b]], sc[b]).wait()

    def compute(b):
        rb_ = rows[b]
        ib_ = idx[b]
        cb_ = cidx[b]

        def edge_group(g, carry):
            sl16 = pl.ds(g * 16, 16)
            ci = ib_[1, sl16] - cbase
            cb_[sl16] = jnp.where(
                jnp.logical_and(ci >= 0, ci < NHALF), ci, NHALF)
            ew16 = plsc.bitcast(ib_[2, sl16], F32)
            for k in range(16):
                e = g * 16 + k
                w = ew16[k]
                rb_[e, pl.ds(0, 16)] = jnp.maximum(
                    rb_[e, pl.ds(0, 16)] + w * vs0, 0.0)
                rb_[e, pl.ds(16, 16)] = jnp.maximum(
                    rb_[e, pl.ds(16, 16)] + w * vs1, 0.0)
            return carry

        lax.fori_loop(0, EC // 16, edge_group, 0)

    for b in range(NBUF - 1):
        idx_load(b, b)
        gather_start(b)

    def outer(g, carry):
        for b in range(NBUF):
            j = g * NBUF + b
            jn = j + (NBUF - 1)
            bn = (b + NBUF - 1) % NBUF
            gather_wait(b)
            compute(b)
            scatter_start(b)

            @pl.when(jn < NCHUNK)
            def _():
                @pl.when(j >= 1)
                def _():
                    scatter_wait(bn)

                idx_load(bn, jn)
                gather_start(bn)

        return carry

    lax.fori_loop(0, NCHUNK // NBUF, outer, 0)

    for b in range(NBUF):
        scatter_wait(b)

    plsc.subcore_barrier()

    @pl.when(c == 0)
    def _():
        pltpu.sync_copy(acc_sh.at[pl.ds(r0, RB)], outa.at[pl.ds(r0, RB)])
        pltpu.sync_copy(cnt_sh.at[pl.ds(s * RBH, RBH)],
                        cnta_out.at[pl.ds(s * RBH, RBH)])

    @pl.when(c == 1)
    def _():
        pltpu.sync_copy(acc_sh.at[pl.ds(r0, RB)], outb.at[pl.ds(r0, RB)])
        pltpu.sync_copy(cnt_sh.at[pl.ds(s * RBH, RBH)],
                        cntb_out.at[pl.ds(s * RBH, RBH)])


_sc_edge_pass = functools.partial(
    pl.kernel,
    out_type=(
        jax.ShapeDtypeStruct((NPAD, 32), F32),
        jax.ShapeDtypeStruct((NPAD, 32), F32),
        jax.ShapeDtypeStruct((NPH, 8), F32),
        jax.ShapeDtypeStruct((NPH, 8), F32),
    ),
    mesh=plsc.VectorSubcoreMesh(core_axis_name="c", subcore_axis_name="s",
                                num_cores=2, num_subcores=NSUB),
    compiler_params=pltpu.CompilerParams(use_tc_tiling_on_sc=False,
                                         needs_layout_passes=False),
    scratch_types=(
        [pltpu.VMEM_SHARED((NPAD, 32), F32),
         pltpu.VMEM_SHARED((NPH, 8), F32),
         pltpu.VMEM((32,), F32),
         pltpu.VMEM((EC, 8), F32)]
        + [pltpu.VMEM((3, EC), jnp.int32) for _ in range(NBUF)]
        + [pltpu.VMEM((EC,), jnp.int32) for _ in range(NBUF)]
        + [pltpu.VMEM((EC, 32), F32) for _ in range(NBUF)]
        + [pltpu.SemaphoreType.DMA for _ in range(3 * NBUF)]
    ),
)(_sc_edge_body)


def _post0_body(h_ref, nb_ref, aa_ref, ab_ref, cnt_ref,
                w2h_ref, w2n_ref, w2ba_ref, w2bb_ref, b2_ref,
                w1a_ref, w1b_ref, b1a_ref, b1b_ref,
                out_ref, ya_ref, yb_ref):
    inv = jnp.dot(1.0 / jnp.maximum(cnt_ref[...], 1.0),
                  jnp.ones((1, H), F32), precision=lax.Precision.HIGHEST)
    z = _mm(aa_ref[...], w2ba_ref[...]) + _mm(ab_ref[...], w2bb_ref[...])
    out = jnp.maximum(
        _mm(h_ref[...], w2h_ref[...]) + _mm(nb_ref[...], w2n_ref[...])
        + z * inv + b2_ref[...], 0.0)
    out_ref[...] = out
    ya_ref[...] = _mm(out, w1a_ref[...]) + b1a_ref[...]
    yb_ref[...] = _mm(out, w1b_ref[...]) + b1b_ref[...]


def _post0(h, nb, acc_a, acc_b, cnt2, w2a, w2b, b22, w1, b12):
    args = (h, nb, acc_a, acc_b, cnt2,
            w2a[:XE], w2a[XE:], w2b[:32], w2b[32:], b22,
            w1[:, :32], w1[:, 32:], b12[:, :32], b12[:, 32:])
    in_specs = [
        pl.BlockSpec((RBLK, XE), lambda i: (i, 0)),
        pl.BlockSpec((RBLK, H), lambda i: (i, 0)),
        pl.BlockSpec((RBLK, 32), lambda i: (i, 0)),
        pl.BlockSpec((RBLK, 32), lambda i: (i, 0)),
        pl.BlockSpec((RBLK, 1), lambda i: (i, 0)),
        pl.BlockSpec((XE, H), lambda i: (0, 0)),
        pl.BlockSpec((H, H), lambda i: (0, 0)),
        pl.BlockSpec((32, H), lambda i: (0, 0)),
        pl.BlockSpec((32, H), lambda i: (0, 0)),
        pl.BlockSpec((1, H), lambda i: (0, 0)),
        pl.BlockSpec((H, 32), lambda i: (0, 0)),
        pl.BlockSpec((H, 32), lambda i: (0, 0)),
        pl.BlockSpec((1, 32), lambda i: (0, 0)),
        pl.BlockSpec((1, 32), lambda i: (0, 0)),
    ]
    out_specs = [
        pl.BlockSpec((RBLK, H), lambda i: (i, 0)),
        pl.BlockSpec((RBLK, 32), lambda i: (i, 0)),
        pl.BlockSpec((RBLK, 32), lambda i: (i, 0)),
    ]
    out_shape = [
        jax.ShapeDtypeStruct((N, H), F32),
        jax.ShapeDtypeStruct((N, 32), F32),
        jax.ShapeDtypeStruct((N, 32), F32),
    ]
    return pl.pallas_call(
        _post0_body, grid=(NBLK,), in_specs=in_specs, out_specs=out_specs,
        out_shape=out_shape,
    )(*args)


def _post1_body(x_ref, aa_ref, ab_ref, cnt_ref,
                w2a_ref, w2ba_ref, w2bb_ref, b2_ref, out_ref):
    inv = jnp.dot(1.0 / jnp.maximum(cnt_ref[...], 1.0),
                  jnp.ones((1, H), F32), precision=lax.Precision.HIGHEST)
    z = _mm(aa_ref[...], w2ba_ref[...]) + _mm(ab_ref[...], w2bb_ref[...])
    out_ref[...] = jnp.maximum(
        _mm(x_ref[...], w2a_ref[...]) + z * inv + b2_ref[...], 0.0)


def _post1(x, acc_a, acc_b, cnt2, w2a, w2b, b22):
    args = (x, acc_a, acc_b, cnt2, w2a, w2b[:32], w2b[32:], b22)
    in_specs = [
        pl.BlockSpec((RBLK, H), lambda i: (i, 0)),
        pl.BlockSpec((RBLK, 32), lambda i: (i, 0)),
        pl.BlockSpec((RBLK, 32), lambda i: (i, 0)),
        pl.BlockSpec((RBLK, 1), lambda i: (i, 0)),
        pl.BlockSpec((H, H), lambda i: (0, 0)),
        pl.BlockSpec((32, H), lambda i: (0, 0)),
        pl.BlockSpec((32, H), lambda i: (0, 0)),
        pl.BlockSpec((1, H), lambda i: (0, 0)),
    ]
    return pl.pallas_call(
        _post1_body, grid=(NBLK,), in_specs=in_specs,
        out_specs=pl.BlockSpec((RBLK, H), lambda i: (i, 0)),
        out_shape=jax.ShapeDtypeStruct((N, H), F32),
    )(*args)


def kernel(x, trans_w, g_edge_index, g_edge_w, edge_index, edge_w, params):
    p = params
    we = p['we']
    be2 = p['be'].reshape(1, XE)
    g0, g1 = p['group']
    gl0, gl1 = p['global']

    gw = (
        g0['mlp1']['w'][:XE], g0['mlp1']['w'][XE:], g0['mlp1']['b'].reshape(1, H),
        g0['mlp2']['w'][:XE], g0['mlp2']['w'][XE:], g0['mlp2']['b'].reshape(1, H),
        g1['mlp1']['w'][:H], g1['mlp1']['w'][H:], g1['mlp1']['b'].reshape(1, H),
        g1['mlp2']['w'][:H], g1['mlp2']['w'][H:], g1['mlp2']['b'].reshape(1, H),
    )
    w1g0 = gl0['mlp1']['w']
    w1h0, w1n0 = w1g0[:XE], w1g0[XE:XE + H]
    v0 = w1g0[XE + H]
    b1g0 = gl0['mlp1']['b'].reshape(1, H)
    w2a0, w2b0 = gl0['mlp2']['w'][:XE + H], gl0['mlp2']['w'][XE + H:]
    b2g0 = gl0['mlp2']['b'].reshape(1, H)
    w1g1 = gl1['mlp1']['w']
    w1a1, v1 = w1g1[:H], w1g1[H]
    b1g1 = gl1['mlp1']['b'].reshape(1, H)
    w2a1, w2b1 = gl1['mlp2']['w'][:H], gl1['mlp2']['w'][H:]
    b2g1 = gl1['mlp2']['b'].reshape(1, H)

    npd = EPAD - E
    erow = jnp.concatenate([edge_index[0], jnp.zeros((npd,), jnp.int32)])
    ecol = jnp.concatenate([edge_index[1], jnp.full((npd,), N, jnp.int32)])
    ewf = jnp.concatenate([edge_w[:, 0], jnp.zeros((npd,), F32)])
    ewbits = lax.bitcast_convert_type(ewf, jnp.int32)
    ei3 = jnp.stack([erow.reshape(NCHT, EC), ecol.reshape(NCHT, EC),
                     ewbits.reshape(NCHT, EC)], axis=1)
    zrows = jnp.zeros((NPAD, 32), F32)
    zcnt = jnp.zeros((NPH, 8), F32)
    ones8 = jnp.ones((EC, 8), F32)

    h, g = _k1a(x, we, be2, trans_w)
    g_out = _k1b(g.reshape(NB * GROUP, XE), g_edge_index, g_edge_w, *gw)
    nb, y0a, y0b = _k1c(h, trans_w, g_out.reshape(NB, GROUP, H),
                        w1h0, w1n0, b1g0)

    acc_a, acc_b, cnta, cntb = _sc_edge_pass(y0a, y0b, ei3, v0,
                                             zrows, zcnt, ones8)
    cnt2 = jnp.concatenate([cnta[:NHALF, 0],
                            cntb[:N - NHALF, 0]]).reshape(N, 1)
    out0, y1a, y1b = _post0(h, nb, acc_a[:N], acc_b[:N], cnt2,
                            w2a0, w2b0, b2g0, w1a1, b1g1)

    acc_a1, acc_b1, _, _ = _sc_edge_pass(y1a, y1b, ei3, v1,
                                         zrows, zcnt, ones8)
    return _post1(out0, acc_a1[:N], acc_b1[:N], cnt2, w2a1, w2b1, b2g1)
